# baseline (speedup 1.0000x reference)
import functools

import jax
import jax.numpy as jnp
from jax import lax
from jax.experimental import pallas as pl
from jax.experimental.pallas import tpu as pltpu
from jax.experimental.pallas import tpu_sc as plsc

NBUF = 4


def _gather_kernel(batch, seq, embed):
    info = plsc.get_sparse_core_info()
    nc, ns = info.num_cores, info.num_subcores
    nw = nc * ns
    rows_per_w = batch // nw
    mesh = plsc.VectorSubcoreMesh(core_axis_name="c", subcore_axis_name="s")

    @functools.partial(
        pl.kernel,
        mesh=mesh,
        compiler_params=pltpu.CompilerParams(use_tc_tiling_on_sc=False),
        out_type=jax.ShapeDtypeStruct((batch, seq, embed), jnp.float32),
        scratch_types=[
            pltpu.VMEM((rows_per_w, seq), jnp.int32),
            pltpu.VMEM((NBUF, seq, embed), jnp.float32),
        ]
        + [pltpu.SemaphoreType.DMA] * (2 * NBUF),
    )
    def k(idx_hbm, tab_hbm, out_hbm, idx_v, rows_v, *sems):
        gsem, osem = sems[:NBUF], sems[NBUF:]
        wid = lax.axis_index("s") * nc + lax.axis_index("c")
        row_base = wid * rows_per_w
        pltpu.sync_copy(idx_hbm.at[pl.ds(row_base, rows_per_w)], idx_v)

        for b in range(NBUF):
            pltpu.async_copy(tab_hbm.at[idx_v.at[b]], rows_v.at[b], gsem[b])

        def body(g, _):
            c0 = g * NBUF
            for b in range(NBUF):
                c = c0 + b
                dst = out_hbm.at[row_base + c]
                pltpu.make_async_copy(
                    tab_hbm.at[idx_v.at[c]], rows_v.at[b], gsem[b]
                ).wait()
                pltpu.async_copy(rows_v.at[b], dst, osem[b])
            for b in range(NBUF):
                c = c0 + b
                dst = out_hbm.at[row_base + c]
                pltpu.make_async_copy(rows_v.at[b], dst, osem[b]).wait()

                @pl.when(c + NBUF < rows_per_w)
                def _():
                    pltpu.async_copy(
                        tab_hbm.at[idx_v.at[c + NBUF]], rows_v.at[b], gsem[b]
                    )

            return ()

        lax.fori_loop(0, rows_per_w // NBUF, body, ())

    return k


def kernel(input_x, table):
    b, s = input_x.shape
    _, embed = table.shape
    return _gather_kernel(b, s, embed)(input_x, table)
